# baseline (speedup 1.0000x reference)
import functools

import jax
import jax.numpy as jnp
from jax import lax
from jax.experimental import pallas as pl
from jax.experimental.pallas import tpu as pltpu
from jax.experimental.pallas import tpu_sc as plsc

_D = 64
_NW = 32
_CB = 256
_T = 50
_B = 16384


def _gather_t(idx_flat, packed):
    b_per_w = _B // _NW
    j_per_t = b_per_w // _CB

    @functools.partial(
        pl.kernel,
        mesh=plsc.VectorSubcoreMesh(core_axis_name="c", subcore_axis_name="s"),
        out_type=jax.ShapeDtypeStruct((_T, _D, _B), jnp.float32),
        compiler_params=pltpu.CompilerParams(
            use_tc_tiling_on_sc=True, needs_layout_passes=False),
        scratch_types=[
            pltpu.VMEM((_CB,), jnp.int32),
            pltpu.VMEM((_CB,), jnp.int32),
            pltpu.VMEM((_CB, 2 * _D), jnp.float32),
            pltpu.VMEM((_CB, 2 * _D), jnp.float32),
            pltpu.VMEM((_D, _CB), jnp.float32),
            pltpu.VMEM((_D, _CB), jnp.float32),
            pltpu.SemaphoreType.DMA,
            pltpu.SemaphoreType.DMA,
            pltpu.SemaphoreType.DMA,
            pltpu.SemaphoreType.DMA,
            pltpu.SemaphoreType.DMA,
            pltpu.SemaphoreType.DMA,
        ],
    )
    def k2(idx_hbm, tbl_hbm, out_hbm,
           ix0, ix1, r0, r1, tr0, tr1,
           is0, is1, gs0, gs1, ws0, ws1):
        wid = lax.axis_index("s") * 2 + lax.axis_index("c")
        b0 = wid * b_per_w
        lane = lax.broadcasted_iota(jnp.int32, (16,), 0)
        ixv, rows, trans = (ix0, ix1), (r0, r1), (tr0, tr1)
        isem, gsem, wsem = (is0, is1), (gs0, gs1), (ws0, ws1)
        n_chunks = _T * j_per_t

        def off_of(c):
            t = c // j_per_t
            j = c % j_per_t
            return t * _B + b0 + j * _CB, t, j

        def i_start(c, b):
            off, _, _ = off_of(c)
            pltpu.async_copy(idx_hbm.at[pl.ds(pl.multiple_of(off, 8), _CB)],
                             ixv[b], isem[b])

        def i_wait(b):
            pltpu.make_async_copy(idx_hbm.at[pl.ds(0, _CB)], ixv[b],
                                  isem[b]).wait()

        def g_start(b):
            pltpu.async_copy(tbl_hbm.at[ixv[b]], rows[b], gsem[b])

        def g_wait(b):
            pltpu.make_async_copy(tbl_hbm.at[ixv[b]], rows[b], gsem[b]).wait()

        def transpose(b):
            def go(g, c):
                rowids = lane + g * 16
                for d0 in range(0, _D, 8):
                    rs = [plsc.load_gather(rows[b], [rowids, dcol[d0 + k]])
                          for k in range(8)]
                    for k in range(8):
                        trans[b][d0 + k, pl.ds(g * 16, 16)] = rs[k]
                return c
            lax.fori_loop(0, _CB // 16, go, 0)

        def w_start(c, b):
            _, t, j = off_of(c)
            pltpu.async_copy(trans[b],
                             out_hbm.at[t, :, pl.ds(b0 + j * _CB, _CB)],
                             wsem[b])

        def w_wait(b):
            pltpu.make_async_copy(trans[b], out_hbm.at[0, :, pl.ds(0, _CB)],
                                  wsem[b]).wait()

        dcol = [jnp.full((16,), d, jnp.int32) for d in range(_D)]
        i_start(0, 0)
        i_wait(0)
        g_start(0)
        i_start(1, 1)

        def step(c, i, b):
            nxt = 1 - b

            @pl.when(c + 1 < n_chunks)
            def _():
                i_wait(nxt)

            g_wait(b)

            @pl.when(c + 1 < n_chunks)
            def _():
                g_start(nxt)

            transpose(b)

            @pl.when(c >= 2)
            def _():
                w_wait(b)

            w_start(c, b)

            @pl.when(c + 2 < n_chunks)
            def _():
                i_start(c + 2, b)

        def body(i, carry):
            step(2 * i, i, 0)
            step(2 * i + 1, i, 1)
            return carry

        lax.fori_loop(0, n_chunks // 2, body, 0)
        w_wait(0)
        w_wait(1)

    return k2(idx_flat, packed)


def kernel(data, ivectors):
    idx_flat = jnp.transpose(data).reshape(-1).astype(jnp.int32)
    packed = jnp.pad(ivectors, ((0, 0), (0, 64)))
    out_t = _gather_t(idx_flat, packed)
    return jnp.transpose(out_t, (2, 0, 1))


# speedup vs baseline: 1.6361x; 1.6361x over previous
import functools

import jax
import jax.numpy as jnp
from jax import lax
from jax.experimental import pallas as pl
from jax.experimental.pallas import tpu as pltpu
from jax.experimental.pallas import tpu_sc as plsc

_D = 64
_NW = 32
_CB = 256
_T = 50
_B = 16384


def _gather_t(idx_flat, packed):
    b_per_w = _B // _NW
    j_per_t = b_per_w // _CB

    @functools.partial(
        pl.kernel,
        mesh=plsc.VectorSubcoreMesh(core_axis_name="c", subcore_axis_name="s"),
        out_type=jax.ShapeDtypeStruct((_T, _D, _B), jnp.float32),
        compiler_params=pltpu.CompilerParams(
            use_tc_tiling_on_sc=True, needs_layout_passes=False),
        scratch_types=[
            pltpu.VMEM((_CB,), jnp.int32),
            pltpu.VMEM((_CB,), jnp.int32),
            pltpu.VMEM((_CB, 2 * _D), jnp.float32),
            pltpu.VMEM((_CB, 2 * _D), jnp.float32),
            pltpu.VMEM((_D, _CB), jnp.float32),
            pltpu.VMEM((_D, _CB), jnp.float32),
            pltpu.SemaphoreType.DMA,
            pltpu.SemaphoreType.DMA,
            pltpu.SemaphoreType.DMA,
            pltpu.SemaphoreType.DMA,
            pltpu.SemaphoreType.DMA,
            pltpu.SemaphoreType.DMA,
        ],
    )
    def k2(idx_hbm, tbl_hbm, out_hbm,
           ix0, ix1, r0, r1, tr0, tr1,
           is0, is1, gs0, gs1, ws0, ws1):
        wid = lax.axis_index("s") * 2 + lax.axis_index("c")
        b0 = wid * b_per_w
        lane = lax.broadcasted_iota(jnp.int32, (16,), 0)
        ixv, rows, trans = (ix0, ix1), (r0, r1), (tr0, tr1)
        isem, gsem, wsem = (is0, is1), (gs0, gs1), (ws0, ws1)
        n_chunks = _T * j_per_t

        def off_of(c):
            t = c // j_per_t
            j = c % j_per_t
            return t * _B + b0 + j * _CB, t, j

        def i_start(c, b):
            off, _, _ = off_of(c)
            pltpu.async_copy(idx_hbm.at[pl.ds(pl.multiple_of(off, 8), _CB)],
                             ixv[b], isem[b])

        def i_wait(b):
            pltpu.make_async_copy(idx_hbm.at[pl.ds(0, _CB)], ixv[b],
                                  isem[b]).wait()

        def g_start(b):
            pltpu.async_copy(tbl_hbm.at[ixv[b]], rows[b], gsem[b])

        def g_wait(b):
            pltpu.make_async_copy(tbl_hbm.at[ixv[b]], rows[b], gsem[b]).wait()

        def transpose(b):
            def go(g, c):
                rowids = lane + g * 16
                for d0 in range(0, _D, 16):
                    for k0 in range(0, 16, 8):
                        cs = [diag[k0 + k] + d0 for k in range(8)]
                        rs = [plsc.load_gather(rows[b], [rowids, cs[k]])
                              for k in range(8)]
                        for k in range(8):
                            plsc.store_scatter(trans[b], [cs[k], rowids], rs[k])
                return c
            lax.fori_loop(0, _CB // 16, go, 0)

        def w_start(c, b):
            _, t, j = off_of(c)
            pltpu.async_copy(trans[b],
                             out_hbm.at[t, :, pl.ds(b0 + j * _CB, _CB)],
                             wsem[b])

        def w_wait(b):
            pltpu.make_async_copy(trans[b], out_hbm.at[0, :, pl.ds(0, _CB)],
                                  wsem[b]).wait()

        diag = [jnp.bitwise_and(lane + k, 15) for k in range(16)]
        i_start(0, 0)
        i_wait(0)
        g_start(0)
        i_start(1, 1)

        def step(c, i, b):
            nxt = 1 - b

            @pl.when(c + 1 < n_chunks)
            def _():
                i_wait(nxt)

            g_wait(b)

            @pl.when(c + 1 < n_chunks)
            def _():
                g_start(nxt)

            transpose(b)

            @pl.when(c >= 2)
            def _():
                w_wait(b)

            w_start(c, b)

            @pl.when(c + 2 < n_chunks)
            def _():
                i_start(c + 2, b)

        def body(i, carry):
            step(2 * i, i, 0)
            step(2 * i + 1, i, 1)
            return carry

        lax.fori_loop(0, n_chunks // 2, body, 0)
        w_wait(0)
        w_wait(1)

    return k2(idx_flat, packed)


def kernel(data, ivectors):
    idx_flat = jnp.transpose(data).reshape(-1).astype(jnp.int32)
    packed = jnp.pad(ivectors, ((0, 0), (0, 64)))
    out_t = _gather_t(idx_flat, packed)
    return jnp.transpose(out_t, (2, 0, 1))
